# Initial kernel scaffold; baseline (speedup 1.0000x reference)
import jax
import jax.numpy as jnp
from jax.experimental import pallas as pl


def kernel(x, tags, W_emb, gamma, beta):
    raise NotImplementedError("write your pallas kernel here")


# speedup vs baseline: 2.7487x; 2.7487x over previous
import jax
import jax.numpy as jnp
from jax.experimental import pallas as pl

B = 16384
D = 1792
EPS = 1e-5
BLK = 256


def _body(t_ref, w_ref, g_ref, b_ref, x_ref, o_ref):
    t = t_ref[...].astype(jnp.float32)
    w0 = w_ref[0:1, :]
    w1 = w_ref[1:2, :]
    q = w0 + t * (w1 - w0)
    h = x_ref[...] + q
    mean = jnp.mean(h, axis=1, keepdims=True)
    c = h - mean
    var = jnp.mean(c * c, axis=1, keepdims=True)
    o_ref[...] = c * jax.lax.rsqrt(var + EPS) * g_ref[...] + b_ref[...]


def kernel(x, tags, W_emb, gamma, beta):
    tcol = tags.reshape(B, 1).astype(jnp.int32)
    g2 = gamma.reshape(1, D)
    b2 = beta.reshape(1, D)
    grid = B // BLK
    return pl.pallas_call(
        _body,
        grid=(grid,),
        in_specs=[
            pl.BlockSpec((BLK, 1), lambda i: (i, 0)),
            pl.BlockSpec((2, D), lambda i: (0, 0)),
            pl.BlockSpec((1, D), lambda i: (0, 0)),
            pl.BlockSpec((1, D), lambda i: (0, 0)),
            pl.BlockSpec((BLK, D), lambda i: (i, 0)),
        ],
        out_specs=pl.BlockSpec((BLK, D), lambda i: (i, 0)),
        out_shape=jax.ShapeDtypeStruct((B, D), jnp.float32),
    )(tcol, W_emb, g2, b2, x)
